# baseline (speedup 1.0000x reference)
import functools

import jax
import jax.numpy as jnp
from jax import lax
from jax.experimental import pallas as pl
from jax.experimental.pallas import tpu as pltpu
from jax.experimental.pallas import tpu_sc as plsc

_BATCH = 4
_SEQ = 8192
_D = 1024
_B = _BATCH * _SEQ
_NC = 2
_NS = 16
_NW = _NC * _NS
_BPW = _B // _NW
_C = 32
_NCHUNK = _BPW // _C
_NBUF = 2


def _emb_body(idx_hbm, table_hbm, out_hbm, idx_v, rows_v, sem0, sem1):
    sems = (sem0, sem1)
    wid = lax.axis_index("s") * _NC + lax.axis_index("c")
    pltpu.sync_copy(idx_hbm.at[wid], idx_v)

    def start_gather(slot, g):
        pltpu.async_copy(table_hbm.at[idx_v.at[g]], rows_v.at[slot], sems[slot])

    def wait_gather(slot, g):
        pltpu.make_async_copy(
            table_hbm.at[idx_v.at[g]], rows_v.at[slot], sems[slot]
        ).wait()

    for b in range(_NBUF):
        start_gather(b, b)

    n_outer = _NCHUNK // _NBUF

    def outer(it, carry):
        for b in range(_NBUF):
            g = it * _NBUF + b
            wait_gather(b, g)
            pltpu.sync_copy(rows_v.at[b], out_hbm.at[wid, g])
            start_gather(b, g + _NBUF)
        return carry

    lax.fori_loop(0, n_outer - 1, outer, 0)

    for b in range(_NBUF):
        g = (n_outer - 1) * _NBUF + b
        wait_gather(b, g)
        pltpu.sync_copy(rows_v.at[b], out_hbm.at[wid, g])


_emb_call = functools.partial(
    pl.kernel,
    out_type=jax.ShapeDtypeStruct((_NW, _NCHUNK, _C, _D), jnp.float32),
    mesh=plsc.VectorSubcoreMesh(core_axis_name="c", subcore_axis_name="s"),
    scratch_types=[
        pltpu.VMEM((_NCHUNK, _C), jnp.int32),
        pltpu.VMEM((_NBUF, _C, _D), jnp.float32),
        pltpu.SemaphoreType.DMA,
        pltpu.SemaphoreType.DMA,
    ],
)(_emb_body)


def kernel(positions, embedding_table):
    idx = positions.astype(jnp.int32).reshape(_NW, _NCHUNK, _C)
    out = _emb_call(idx, embedding_table)
    return out.reshape(_BATCH, _SEQ, _D)
